# baseline (speedup 1.0000x reference)
import functools

import jax
import jax.numpy as jnp
from jax import lax
from jax.experimental import pallas as pl
from jax.experimental.pallas import tpu as pltpu
from jax.experimental.pallas import tpu_sc as plsc

_FULL = (100000, 128)
_N = _FULL[0] * _FULL[1]
_NC = 2
_NS = 16
_NW = _NC * _NS
_PER_W = _N // _NW
_CHUNK = 10_000
_NCHUNK = _PER_W // _CHUNK
_SP = 9_200
_HB = _CHUNK - _SP

_BANK = 1_280_000
_BANK_SLICE = _BANK // _NS

_mesh = plsc.VectorSubcoreMesh(core_axis_name="c", subcore_axis_name="s")


@functools.partial(
    pl.kernel,
    mesh=_mesh,
    out_type=jax.ShapeDtypeStruct((_N,), jnp.float32),
    scratch_types=[
        pltpu.VMEM((_CHUNK,), jnp.int32),
        pltpu.VMEM((_CHUNK,), jnp.int32),
        pltpu.VMEM((_CHUNK,), jnp.float32),
        pltpu.VMEM((_CHUNK,), jnp.float32),
        pltpu.VMEM_SHARED((_BANK,), jnp.float32),
        pltpu.SemaphoreType.DMA,
        pltpu.SemaphoreType.DMA,
        pltpu.SemaphoreType.DMA,
        pltpu.SemaphoreType.DMA,
        pltpu.SemaphoreType.DMA,
        pltpu.SemaphoreType.DMA,
        pltpu.SemaphoreType.DMA,
        pltpu.SemaphoreType.DMA,
    ],
)
def _gather_kernel(bank_hbm, idx_hbm, out_hbm, idx_v0, idx_v1, val_v0, val_v1,
                   bank_sp, sa0, sa1, sb0, sb1, sh0, sh1, sc0, sc1):
    sid = lax.axis_index("s")
    wid = sid * _NC + lax.axis_index("c")
    base = wid * _PER_W
    idx_v = (idx_v0, idx_v1)
    val_v = (val_v0, val_v1)
    sa = (sa0, sa1)
    sb = (sb0, sb1)
    sh = (sh0, sh1)
    sc = (sc0, sc1)

    pltpu.sync_copy(bank_hbm.at[pl.ds(sid * _BANK_SLICE, _BANK_SLICE)],
                    bank_sp.at[pl.ds(sid * _BANK_SLICE, _BANK_SLICE)])
    plsc.subcore_barrier()

    def start_a(g, b):
        pltpu.async_copy(idx_hbm.at[pl.ds(base + g * _CHUNK, _CHUNK)],
                         idx_v[b], sa[b])

    def wait_a(b):
        pltpu.make_async_copy(idx_hbm.at[pl.ds(base, _CHUNK)],
                              idx_v[b], sa[b]).wait()

    def start_b(b):
        pltpu.async_copy(bank_sp.at[idx_v[b].at[pl.ds(0, _SP)]],
                         val_v[b].at[pl.ds(0, _SP)], sb[b])
        pltpu.async_copy(bank_hbm.at[idx_v[b].at[pl.ds(_SP, _HB)]],
                         val_v[b].at[pl.ds(_SP, _HB)], sh[b])

    def wait_b(b):
        pltpu.make_async_copy(bank_sp.at[idx_v[b].at[pl.ds(0, _SP)]],
                              val_v[b].at[pl.ds(0, _SP)], sb[b]).wait()
        pltpu.make_async_copy(bank_hbm.at[idx_v[b].at[pl.ds(_SP, _HB)]],
                              val_v[b].at[pl.ds(_SP, _HB)], sh[b]).wait()

    def start_c(g, b):
        pltpu.async_copy(val_v[b],
                         out_hbm.at[pl.ds(base + g * _CHUNK, _CHUNK)], sc[b])

    def wait_c(b):
        pltpu.make_async_copy(val_v[b],
                              out_hbm.at[pl.ds(base, _CHUNK)], sc[b]).wait()

    start_a(0, 0)
    start_a(1, 1)
    wait_a(0)
    start_b(0)
    wait_b(0)
    start_c(0, 0)
    start_a(2, 0)
    wait_a(1)
    start_b(1)

    def body(j, carry):
        g0 = 2 * j
        g1 = g0 + 1
        wait_b(1)
        start_c(g1 - 2, 1)
        start_a(g1, 1)
        wait_a(0)
        wait_c(0)
        start_b(0)
        wait_b(0)
        start_c(g0, 0)

        @pl.when(g1 + 1 < _NCHUNK)
        def _():
            start_a(g1 + 1, 0)

        wait_a(1)
        wait_c(1)
        start_b(1)
        return carry

    lax.fori_loop(1, _NCHUNK // 2, body, 0)

    wait_b(1)
    start_c(_NCHUNK - 1, 1)
    wait_c(0)
    wait_c(1)


def kernel(bank, weight_alloc):
    idx = weight_alloc.reshape(-1).astype(jnp.int32)
    out = _gather_kernel(bank, idx)
    return out.reshape(_FULL)


# speedup vs baseline: 1.0422x; 1.0422x over previous
import functools

import jax
import jax.numpy as jnp
from jax import lax
from jax.experimental import pallas as pl
from jax.experimental.pallas import tpu as pltpu
from jax.experimental.pallas import tpu_sc as plsc

_FULL = (100000, 128)
_N = _FULL[0] * _FULL[1]
_NC = 2
_NS = 16
_NW = _NC * _NS
_PER_W = _N // _NW
_CHUNK = 10_000
_NCHUNK = _PER_W // _CHUNK
_SP = 8_800
_HB = _CHUNK - _SP

_BANK = 1_280_000
_BANK_SLICE = _BANK // _NS

_mesh = plsc.VectorSubcoreMesh(core_axis_name="c", subcore_axis_name="s")


@functools.partial(
    pl.kernel,
    mesh=_mesh,
    out_type=jax.ShapeDtypeStruct((_N,), jnp.float32),
    scratch_types=[
        pltpu.VMEM((_CHUNK,), jnp.int32),
        pltpu.VMEM((_CHUNK,), jnp.int32),
        pltpu.VMEM((_CHUNK,), jnp.float32),
        pltpu.VMEM((_CHUNK,), jnp.float32),
        pltpu.VMEM_SHARED((_BANK,), jnp.float32),
        pltpu.SemaphoreType.DMA,
        pltpu.SemaphoreType.DMA,
        pltpu.SemaphoreType.DMA,
        pltpu.SemaphoreType.DMA,
        pltpu.SemaphoreType.DMA,
        pltpu.SemaphoreType.DMA,
        pltpu.SemaphoreType.DMA,
        pltpu.SemaphoreType.DMA,
    ],
)
def _gather_kernel(bank_hbm, idx_hbm, out_hbm, idx_v0, idx_v1, val_v0, val_v1,
                   bank_sp, sa0, sa1, sb0, sb1, sh0, sh1, sc0, sc1):
    sid = lax.axis_index("s")
    wid = sid * _NC + lax.axis_index("c")
    base = wid * _PER_W
    idx_v = (idx_v0, idx_v1)
    val_v = (val_v0, val_v1)
    sa = (sa0, sa1)
    sb = (sb0, sb1)
    sh = (sh0, sh1)
    sc = (sc0, sc1)

    def start_a(g, b):
        pltpu.async_copy(idx_hbm.at[pl.ds(base + g * _CHUNK, _CHUNK)],
                         idx_v[b], sa[b])

    def wait_a(b):
        pltpu.make_async_copy(idx_hbm.at[pl.ds(base, _CHUNK)],
                              idx_v[b], sa[b]).wait()

    def start_b(b):
        pltpu.async_copy(bank_sp.at[idx_v[b].at[pl.ds(0, _SP)]],
                         val_v[b].at[pl.ds(0, _SP)], sb[b])
        pltpu.async_copy(bank_hbm.at[idx_v[b].at[pl.ds(_SP, _HB)]],
                         val_v[b].at[pl.ds(_SP, _HB)], sh[b])

    def wait_b(b):
        pltpu.make_async_copy(bank_sp.at[idx_v[b].at[pl.ds(0, _SP)]],
                              val_v[b].at[pl.ds(0, _SP)], sb[b]).wait()
        pltpu.make_async_copy(bank_hbm.at[idx_v[b].at[pl.ds(_SP, _HB)]],
                              val_v[b].at[pl.ds(_SP, _HB)], sh[b]).wait()

    def start_c(g, b):
        pltpu.async_copy(val_v[b],
                         out_hbm.at[pl.ds(base + g * _CHUNK, _CHUNK)], sc[b])

    def wait_c(b):
        pltpu.make_async_copy(val_v[b],
                              out_hbm.at[pl.ds(base, _CHUNK)], sc[b]).wait()

    start_a(0, 0)
    start_a(1, 1)
    pltpu.sync_copy(bank_hbm.at[pl.ds(sid * _BANK_SLICE, _BANK_SLICE)],
                    bank_sp.at[pl.ds(sid * _BANK_SLICE, _BANK_SLICE)])
    plsc.subcore_barrier()
    wait_a(0)
    start_b(0)
    wait_b(0)
    start_c(0, 0)
    start_a(2, 0)
    wait_a(1)
    start_b(1)

    def body(j, carry):
        g0 = 2 * j
        g1 = g0 + 1
        wait_b(1)
        start_c(g1 - 2, 1)
        start_a(g1, 1)
        wait_a(0)
        wait_c(0)
        start_b(0)
        wait_b(0)
        start_c(g0, 0)

        @pl.when(g1 + 1 < _NCHUNK)
        def _():
            start_a(g1 + 1, 0)

        wait_a(1)
        wait_c(1)
        start_b(1)
        return carry

    lax.fori_loop(1, _NCHUNK // 2, body, 0)

    wait_b(1)
    start_c(_NCHUNK - 1, 1)
    wait_c(0)
    wait_c(1)


def kernel(bank, weight_alloc):
    idx = weight_alloc.reshape(-1).astype(jnp.int32)
    out = _gather_kernel(bank, idx)
    return out.reshape(_FULL)


# speedup vs baseline: 1.0445x; 1.0022x over previous
import functools

import jax
import jax.numpy as jnp
from jax import lax
from jax.experimental import pallas as pl
from jax.experimental.pallas import tpu as pltpu
from jax.experimental.pallas import tpu_sc as plsc

_FULL = (100000, 128)
_N = _FULL[0] * _FULL[1]
_NC = 2
_NS = 16
_NW = _NC * _NS
_PER_W = _N // _NW
_CHUNK = 10_000
_NCHUNK = _PER_W // _CHUNK
_SP = 8_800
_HB = _CHUNK - _SP

_BANK = 1_280_000
_BANK_SLICE = _BANK // _NS

_mesh = plsc.VectorSubcoreMesh(core_axis_name="c", subcore_axis_name="s")


@functools.partial(
    pl.kernel,
    mesh=_mesh,
    out_type=jax.ShapeDtypeStruct((_N,), jnp.float32),
    scratch_types=[
        pltpu.VMEM((_CHUNK,), jnp.int32),
        pltpu.VMEM((_CHUNK,), jnp.int32),
        pltpu.VMEM((_CHUNK,), jnp.float32),
        pltpu.VMEM((_CHUNK,), jnp.float32),
        pltpu.VMEM_SHARED((_BANK,), jnp.float32),
        pltpu.SemaphoreType.DMA,
        pltpu.SemaphoreType.DMA,
        pltpu.SemaphoreType.DMA,
        pltpu.SemaphoreType.DMA,
        pltpu.SemaphoreType.DMA,
        pltpu.SemaphoreType.DMA,
        pltpu.SemaphoreType.DMA,
        pltpu.SemaphoreType.DMA,
        pltpu.SemaphoreType.DMA,
        pltpu.SemaphoreType.DMA,
    ],
)
def _gather_kernel(bank_hbm, idx_hbm, out_hbm, idx_v0, idx_v1, val_v0, val_v1,
                   bank_sp, sa0, sa1, sb0, sb1, sb20, sb21, sh0, sh1, sc0, sc1):
    sid = lax.axis_index("s")
    wid = sid * _NC + lax.axis_index("c")
    base = wid * _PER_W
    idx_v = (idx_v0, idx_v1)
    val_v = (val_v0, val_v1)
    sa = (sa0, sa1)
    sb = (sb0, sb1)
    sb2 = (sb20, sb21)
    sh = (sh0, sh1)
    sc = (sc0, sc1)

    def start_a(g, b):
        pltpu.async_copy(idx_hbm.at[pl.ds(base + g * _CHUNK, _CHUNK)],
                         idx_v[b], sa[b])

    def wait_a(b):
        pltpu.make_async_copy(idx_hbm.at[pl.ds(base, _CHUNK)],
                              idx_v[b], sa[b]).wait()

    _H1 = _SP // 2

    def start_b(b):
        pltpu.async_copy(bank_sp.at[idx_v[b].at[pl.ds(0, _H1)]],
                         val_v[b].at[pl.ds(0, _H1)], sb[b])
        pltpu.async_copy(bank_sp.at[idx_v[b].at[pl.ds(_H1, _SP - _H1)]],
                         val_v[b].at[pl.ds(_H1, _SP - _H1)], sb2[b])
        pltpu.async_copy(bank_hbm.at[idx_v[b].at[pl.ds(_SP, _HB)]],
                         val_v[b].at[pl.ds(_SP, _HB)], sh[b])

    def wait_b(b):
        pltpu.make_async_copy(bank_sp.at[idx_v[b].at[pl.ds(0, _H1)]],
                              val_v[b].at[pl.ds(0, _H1)], sb[b]).wait()
        pltpu.make_async_copy(bank_sp.at[idx_v[b].at[pl.ds(_H1, _SP - _H1)]],
                              val_v[b].at[pl.ds(_H1, _SP - _H1)], sb2[b]).wait()
        pltpu.make_async_copy(bank_hbm.at[idx_v[b].at[pl.ds(_SP, _HB)]],
                              val_v[b].at[pl.ds(_SP, _HB)], sh[b]).wait()

    def start_c(g, b):
        pltpu.async_copy(val_v[b],
                         out_hbm.at[pl.ds(base + g * _CHUNK, _CHUNK)], sc[b])

    def wait_c(b):
        pltpu.make_async_copy(val_v[b],
                              out_hbm.at[pl.ds(base, _CHUNK)], sc[b]).wait()

    start_a(0, 0)
    start_a(1, 1)
    pltpu.sync_copy(bank_hbm.at[pl.ds(sid * _BANK_SLICE, _BANK_SLICE)],
                    bank_sp.at[pl.ds(sid * _BANK_SLICE, _BANK_SLICE)])
    plsc.subcore_barrier()
    wait_a(0)
    start_b(0)
    wait_b(0)
    start_c(0, 0)
    start_a(2, 0)
    wait_a(1)
    start_b(1)

    def body(j, carry):
        g0 = 2 * j
        g1 = g0 + 1
        wait_b(1)
        start_c(g1 - 2, 1)
        start_a(g1, 1)
        wait_a(0)
        wait_c(0)
        start_b(0)
        wait_b(0)
        start_c(g0, 0)

        @pl.when(g1 + 1 < _NCHUNK)
        def _():
            start_a(g1 + 1, 0)

        wait_a(1)
        wait_c(1)
        start_b(1)
        return carry

    lax.fori_loop(1, _NCHUNK // 2, body, 0)

    wait_b(1)
    start_c(_NCHUNK - 1, 1)
    wait_c(0)
    wait_c(1)


def kernel(bank, weight_alloc):
    idx = weight_alloc.reshape(-1).astype(jnp.int32)
    out = _gather_kernel(bank, idx)
    return out.reshape(_FULL)
